# baseline (speedup 1.0000x reference)
import functools

import jax
import jax.numpy as jnp
import numpy as np
from jax.experimental import pallas as pl
from jax.experimental.pallas import tpu as pltpu
from jax.sharding import Mesh, PartitionSpec as P

B, T, D = 32, 1024, 256
K = 1024
N = B * T
COMMITMENT_COST = 0.25

BM = 512


def _vq_body(z_ref, zsq_ref, esq_ref, e2_ref, e_ref, enc_ref, qst_ref, loss_ref):
    z = z_ref[...]
    mm2 = jax.lax.dot_general(
        z, e2_ref[...], (((1,), (1,)), ((), ())),
        preferred_element_type=jnp.float32,
    )
    dist = (zsq_ref[...] + esq_ref[...]) - mm2
    idx = jnp.argmin(dist, axis=1)
    iota = jax.lax.broadcasted_iota(jnp.int32, (BM, K), 1)
    enc = (iota == idx[:, None]).astype(jnp.float32)
    enc_ref[...] = enc
    q = jax.lax.dot_general(
        enc, e_ref[...], (((1,), (0,)), ((), ())),
        preferred_element_type=jnp.float32,
    )
    qst_ref[...] = z + (q - z)
    loss_ref[0, 0, 0] = jnp.sum(jnp.min(dist, axis=1))


def _vq_shard(flat, zsq, esq, e2, e):
    n_local = flat.shape[0]
    grid = n_local // BM
    return pl.pallas_call(
        _vq_body,
        grid=(grid,),
        in_specs=[
            pl.BlockSpec((BM, D), lambda i: (i, 0)),
            pl.BlockSpec((BM, 1), lambda i: (i, 0)),
            pl.BlockSpec((1, K), lambda i: (0, 0)),
            pl.BlockSpec((K, D), lambda i: (0, 0)),
            pl.BlockSpec((K, D), lambda i: (0, 0)),
        ],
        out_specs=[
            pl.BlockSpec((BM, K), lambda i: (i, 0)),
            pl.BlockSpec((BM, D), lambda i: (i, 0)),
            pl.BlockSpec((1, 1, 1), lambda i: (i, 0, 0), memory_space=pltpu.SMEM),
        ],
        out_shape=[
            jax.ShapeDtypeStruct((n_local, K), jnp.float32),
            jax.ShapeDtypeStruct((n_local, D), jnp.float32),
            jax.ShapeDtypeStruct((grid, 1, 1), jnp.float32),
        ],
        compiler_params=pltpu.CompilerParams(
            dimension_semantics=("parallel",),
        ),
    )(flat, zsq, esq, e2, e)


@jax.jit
def kernel(inputs, embedding_weight):
    flat = inputs.reshape(N, D)
    zsq = jnp.sum(flat ** 2, axis=1, keepdims=True)
    esq = jnp.sum(embedding_weight ** 2, axis=1)[None, :]
    e2 = embedding_weight * 2.0

    devs = jax.devices()
    ndev = len(devs) if (N // BM) % len(devs) == 0 else 1
    mesh = Mesh(np.array(devs[:ndev]), ("dp",))
    sharded = jax.shard_map(
        _vq_shard,
        mesh=mesh,
        in_specs=(P("dp"), P("dp"), P(), P(), P()),
        out_specs=(P("dp"), P("dp"), P("dp")),
        check_vma=False,
    )
    enc, qst, loss_parts = sharded(flat, zsq, esq, e2, embedding_weight)

    loss = COMMITMENT_COST * (jnp.sum(loss_parts) / (N * D))
    return qst.reshape(inputs.shape), loss, enc


# speedup vs baseline: 3.1156x; 3.1156x over previous
import functools

import jax
import jax.numpy as jnp
import numpy as np
from jax.experimental import pallas as pl
from jax.experimental.pallas import tpu as pltpu
from jax.sharding import Mesh, PartitionSpec as P

B, T, D = 32, 1024, 256
K = 1024
N = B * T
COMMITMENT_COST = 0.25

BM = 512


def _vq_body(z_ref, zsq_ref, esq_ref, e2_ref, e_ref, enc_ref, qst_ref, loss_ref):
    z = z_ref[...]
    mm2 = jax.lax.dot_general(
        z, e2_ref[...], (((1,), (1,)), ((), ())),
        preferred_element_type=jnp.float32,
    )
    dist = (zsq_ref[...] + esq_ref[...]) - mm2
    idx = jnp.argmin(dist, axis=1)
    iota = jax.lax.broadcasted_iota(jnp.int32, (BM, K), 1)
    onehot = iota == idx[:, None]
    enc_ref[...] = onehot.astype(jnp.float32)
    q = jax.lax.dot_general(
        onehot.astype(jnp.bfloat16), e_ref[...].astype(jnp.bfloat16),
        (((1,), (0,)), ((), ())),
        preferred_element_type=jnp.float32,
    )
    qst_ref[...] = z + (q - z)
    loss_ref[0, 0, 0] = jnp.sum(jnp.min(dist, axis=1))


def _vq_shard(flat, zsq, esq, e2, e):
    n_local = flat.shape[0]
    grid = n_local // BM
    return pl.pallas_call(
        _vq_body,
        grid=(grid,),
        in_specs=[
            pl.BlockSpec((BM, D), lambda i: (i, 0)),
            pl.BlockSpec((BM, 1), lambda i: (i, 0)),
            pl.BlockSpec((1, K), lambda i: (0, 0)),
            pl.BlockSpec((K, D), lambda i: (0, 0)),
            pl.BlockSpec((K, D), lambda i: (0, 0)),
        ],
        out_specs=[
            pl.BlockSpec((BM, K), lambda i: (i, 0)),
            pl.BlockSpec((BM, D), lambda i: (i, 0)),
            pl.BlockSpec((1, 1, 1), lambda i: (i, 0, 0), memory_space=pltpu.SMEM),
        ],
        out_shape=[
            jax.ShapeDtypeStruct((n_local, K), jnp.float32),
            jax.ShapeDtypeStruct((n_local, D), jnp.float32),
            jax.ShapeDtypeStruct((grid, 1, 1), jnp.float32),
        ],
        compiler_params=pltpu.CompilerParams(
            dimension_semantics=("parallel",),
        ),
    )(flat, zsq, esq, e2, e)


@jax.jit
def kernel(inputs, embedding_weight):
    flat = inputs.reshape(N, D)
    zsq = jnp.sum(flat ** 2, axis=1, keepdims=True)
    esq = jnp.sum(embedding_weight ** 2, axis=1)[None, :]
    e2 = embedding_weight * 2.0

    enc, qst, loss_parts = _vq_shard(flat, zsq, esq, e2, embedding_weight)

    loss = COMMITMENT_COST * (jnp.sum(loss_parts) / (N * D))
    return qst.reshape(inputs.shape), loss, enc


# speedup vs baseline: 3.5261x; 1.1318x over previous
import functools

import jax
import jax.numpy as jnp
import numpy as np
from jax.experimental import pallas as pl
from jax.experimental.pallas import tpu as pltpu
from jax.sharding import Mesh, PartitionSpec as P

B, T, D = 32, 1024, 256
K = 1024
N = B * T
COMMITMENT_COST = 0.25

BM = 1024


def _vq_body(z_ref, zsq_ref, esq_ref, e2_ref, e_ref, enc_ref, qst_ref, loss_ref):
    z = z_ref[...]
    mm2 = jax.lax.dot_general(
        z, e2_ref[...], (((1,), (1,)), ((), ())),
        preferred_element_type=jnp.float32,
    )
    dist = (zsq_ref[...] + esq_ref[...]) - mm2
    idx = jnp.argmin(dist, axis=1)
    iota = jax.lax.broadcasted_iota(jnp.int32, (BM, K), 1)
    onehot = iota == idx[:, None]
    enc_ref[...] = onehot.astype(jnp.float32)
    q = jax.lax.dot_general(
        onehot.astype(jnp.bfloat16), e_ref[...].astype(jnp.bfloat16),
        (((1,), (0,)), ((), ())),
        preferred_element_type=jnp.float32,
    )
    qst_ref[...] = z + (q - z)
    loss_ref[0, 0, 0] = jnp.sum(jnp.min(dist, axis=1))


def _vq_shard(flat, zsq, esq, e2, e):
    n_local = flat.shape[0]
    grid = n_local // BM
    return pl.pallas_call(
        _vq_body,
        grid=(grid,),
        in_specs=[
            pl.BlockSpec((BM, D), lambda i: (i, 0)),
            pl.BlockSpec((BM, 1), lambda i: (i, 0)),
            pl.BlockSpec((1, K), lambda i: (0, 0)),
            pl.BlockSpec((K, D), lambda i: (0, 0)),
            pl.BlockSpec((K, D), lambda i: (0, 0)),
        ],
        out_specs=[
            pl.BlockSpec((BM, K), lambda i: (i, 0)),
            pl.BlockSpec((BM, D), lambda i: (i, 0)),
            pl.BlockSpec((1, 1, 1), lambda i: (i, 0, 0), memory_space=pltpu.SMEM),
        ],
        out_shape=[
            jax.ShapeDtypeStruct((n_local, K), jnp.float32),
            jax.ShapeDtypeStruct((n_local, D), jnp.float32),
            jax.ShapeDtypeStruct((grid, 1, 1), jnp.float32),
        ],
        compiler_params=pltpu.CompilerParams(
            dimension_semantics=("parallel",),
        ),
    )(flat, zsq, esq, e2, e)


@jax.jit
def kernel(inputs, embedding_weight):
    flat = inputs.reshape(N, D)
    zsq = jnp.sum(flat ** 2, axis=1, keepdims=True)
    esq = jnp.sum(embedding_weight ** 2, axis=1)[None, :]
    e2 = embedding_weight * 2.0

    enc, qst, loss_parts = _vq_shard(flat, zsq, esq, e2, embedding_weight)

    loss = COMMITMENT_COST * (jnp.sum(loss_parts) / (N * D))
    return qst.reshape(inputs.shape), loss, enc


# speedup vs baseline: 3.9622x; 1.1237x over previous
import functools

import jax
import jax.numpy as jnp
import numpy as np
from jax.experimental import pallas as pl
from jax.experimental.pallas import tpu as pltpu
from jax.sharding import Mesh, PartitionSpec as P

B, T, D = 32, 1024, 256
K = 1024
N = B * T
COMMITMENT_COST = 0.25

BM = 2048


def _vq_body(z_ref, zsq_ref, esq_ref, e2_ref, e_ref, enc_ref, qst_ref, loss_ref):
    z = z_ref[...]
    mm2 = jax.lax.dot_general(
        z, e2_ref[...], (((1,), (1,)), ((), ())),
        preferred_element_type=jnp.float32,
    )
    dist = (zsq_ref[...] + esq_ref[...]) - mm2
    idx = jnp.argmin(dist, axis=1)
    iota = jax.lax.broadcasted_iota(jnp.int32, (BM, K), 1)
    onehot = iota == idx[:, None]
    enc_ref[...] = onehot.astype(jnp.float32)
    q = jax.lax.dot_general(
        onehot.astype(jnp.bfloat16), e_ref[...].astype(jnp.bfloat16),
        (((1,), (0,)), ((), ())),
        preferred_element_type=jnp.float32,
    )
    qst_ref[...] = z + (q - z)
    loss_ref[0, 0, 0] = jnp.sum(jnp.min(dist, axis=1))


def _vq_shard(flat, zsq, esq, e2, e):
    n_local = flat.shape[0]
    grid = n_local // BM
    return pl.pallas_call(
        _vq_body,
        grid=(grid,),
        in_specs=[
            pl.BlockSpec((BM, D), lambda i: (i, 0)),
            pl.BlockSpec((BM, 1), lambda i: (i, 0)),
            pl.BlockSpec((1, K), lambda i: (0, 0)),
            pl.BlockSpec((K, D), lambda i: (0, 0)),
            pl.BlockSpec((K, D), lambda i: (0, 0)),
        ],
        out_specs=[
            pl.BlockSpec((BM, K), lambda i: (i, 0)),
            pl.BlockSpec((BM, D), lambda i: (i, 0)),
            pl.BlockSpec((1, 1, 1), lambda i: (i, 0, 0), memory_space=pltpu.SMEM),
        ],
        out_shape=[
            jax.ShapeDtypeStruct((n_local, K), jnp.float32),
            jax.ShapeDtypeStruct((n_local, D), jnp.float32),
            jax.ShapeDtypeStruct((grid, 1, 1), jnp.float32),
        ],
        compiler_params=pltpu.CompilerParams(
            dimension_semantics=("parallel",),
        ),
    )(flat, zsq, esq, e2, e)


@jax.jit
def kernel(inputs, embedding_weight):
    flat = inputs.reshape(N, D)
    zsq = jnp.sum(flat ** 2, axis=1, keepdims=True)
    esq = jnp.sum(embedding_weight ** 2, axis=1)[None, :]
    e2 = embedding_weight * 2.0

    enc, qst, loss_parts = _vq_shard(flat, zsq, esq, e2, embedding_weight)

    loss = COMMITMENT_COST * (jnp.sum(loss_parts) / (N * D))
    return qst.reshape(inputs.shape), loss, enc
